# Initial kernel scaffold; baseline (speedup 1.0000x reference)
import functools

import jax
import jax.numpy as jnp
from jax import lax
from jax.experimental import pallas as pl
from jax.experimental.pallas import tpu as pltpu
from jax.experimental.pallas import tpu_sc as plsc

_NC = 2
_NS = 16
_C = 128
_B = 1000


def _seg_sum_sc(y, src, dst, zrow):
    N, F = y.shape
    E = src.shape[0]
    num_chunks = E // _C
    rpt = N // _NS
    cpt = -(-num_chunks // (_NC * _NS))
    mesh = plsc.VectorSubcoreMesh(core_axis_name="c", subcore_axis_name="s")

    @functools.partial(
        pl.kernel,
        out_type=jax.ShapeDtypeStruct((_NC * N, F), jnp.float32),
        mesh=mesh,
        scratch_types=[
            pltpu.VMEM((_C,), jnp.int32),
            pltpu.VMEM((_C,), jnp.int32),
            pltpu.VMEM((_C, F), jnp.float32),
            pltpu.VMEM_SHARED((N, F), jnp.float32),
            pltpu.SemaphoreType.DMA,
        ],
    )
    def agg(y_hbm, src_hbm, dst_hbm, z_hbm, out_hbm, src_v, dst_v, rows_v,
            acc_sh, sem):
        cid = lax.axis_index("c")
        sid = lax.axis_index("s")
        wid = sid * _NC + cid
        row0 = sid * rpt
        pltpu.sync_copy(z_hbm, acc_sh.at[pl.ds(row0, rpt)])
        plsc.subcore_barrier()
        start_chunk = wid * cpt

        def body(j, carry):
            chunk = start_chunk + j

            @pl.when(chunk < num_chunks)
            def _():
                base = chunk * _C
                pltpu.sync_copy(src_hbm.at[pl.ds(base, _C)], src_v)
                pltpu.sync_copy(dst_hbm.at[pl.ds(base, _C)], dst_v)
                pltpu.async_copy(y_hbm.at[src_v], rows_v, sem).wait()
                pltpu.sync_copy(rows_v, acc_sh.at[dst_v], add=True)

            return carry

        lax.fori_loop(0, cpt, body, 0)
        plsc.subcore_barrier()
        pltpu.sync_copy(acc_sh.at[pl.ds(row0, rpt)],
                        out_hbm.at[pl.ds(cid * N + row0, rpt)])

    return agg(y, src, dst, zrow)


def _mm_tc(x, W):
    N, D = x.shape
    H = W.shape[1]

    def body(x_ref, w_ref, o_ref):
        o_ref[...] = jnp.dot(x_ref[...], w_ref[...],
                             preferred_element_type=jnp.float32)

    return pl.pallas_call(
        body,
        grid=(N // _B,),
        in_specs=[
            pl.BlockSpec((_B, D), lambda i: (i, 0)),
            pl.BlockSpec((D, H), lambda i: (0, 0)),
        ],
        out_specs=pl.BlockSpec((_B, H), lambda i: (i, 0)),
        out_shape=jax.ShapeDtypeStruct((N, H), jnp.float32),
    )(x, W)


def _mid_tc(opeps, y, p0, p1, ba, Wb, bb, Wnext):
    N, H = y.shape
    H2 = Wnext.shape[1]

    def body(e_ref, y_ref, p0_ref, p1_ref, ba_ref, wb_ref, bb_ref, wn_ref,
             o_ref):
        e = e_ref[0]
        t = jnp.maximum(e * y_ref[...] + p0_ref[...] + p1_ref[...]
                        + ba_ref[...], 0.0)
        h = jnp.dot(t, wb_ref[...], preferred_element_type=jnp.float32)
        h = jnp.maximum(h + bb_ref[...], 0.0)
        o_ref[...] = jnp.dot(h, wn_ref[...],
                             preferred_element_type=jnp.float32)

    return pl.pallas_call(
        body,
        grid=(N // _B,),
        in_specs=[
            pl.BlockSpec(memory_space=pltpu.SMEM),
            pl.BlockSpec((_B, H), lambda i: (i, 0)),
            pl.BlockSpec((_B, H), lambda i: (i, 0)),
            pl.BlockSpec((_B, H), lambda i: (i, 0)),
            pl.BlockSpec((1, H), lambda i: (0, 0)),
            pl.BlockSpec((H, H), lambda i: (0, 0)),
            pl.BlockSpec((1, H), lambda i: (0, 0)),
            pl.BlockSpec((H, H2), lambda i: (0, 0)),
        ],
        out_specs=pl.BlockSpec((_B, H2), lambda i: (i, 0)),
        out_shape=jax.ShapeDtypeStruct((N, H2), jnp.float32),
    )(opeps, y, p0, p1, ba, Wb, bb, Wnext)


def _head_tc(opeps, z, q0, q1, ba, Wb, bb, Wout, bout):
    N, H = z.shape
    O = Wout.shape[1]

    def body(e_ref, z_ref, q0_ref, q1_ref, ba_ref, wb_ref, bb_ref, wo_ref,
             bo_ref, o_ref):
        e = e_ref[0]
        t = jnp.maximum(e * z_ref[...] + q0_ref[...] + q1_ref[...]
                        + ba_ref[...], 0.0)
        h = jnp.dot(t, wb_ref[...], preferred_element_type=jnp.float32)
        h = jnp.maximum(h + bb_ref[...], 0.0)
        logits = jnp.dot(h, wo_ref[...],
                         preferred_element_type=jnp.float32) + bo_ref[...]
        m = jnp.max(logits, axis=-1, keepdims=True)
        s = logits - m
        lse = jnp.log(jnp.sum(jnp.exp(s), axis=-1, keepdims=True))
        o_ref[...] = s - lse

    return pl.pallas_call(
        body,
        grid=(N // _B,),
        in_specs=[
            pl.BlockSpec(memory_space=pltpu.SMEM),
            pl.BlockSpec((_B, H), lambda i: (i, 0)),
            pl.BlockSpec((_B, H), lambda i: (i, 0)),
            pl.BlockSpec((_B, H), lambda i: (i, 0)),
            pl.BlockSpec((1, H), lambda i: (0, 0)),
            pl.BlockSpec((H, H), lambda i: (0, 0)),
            pl.BlockSpec((1, H), lambda i: (0, 0)),
            pl.BlockSpec((H, O), lambda i: (0, 0)),
            pl.BlockSpec((1, O), lambda i: (0, 0)),
        ],
        out_specs=pl.BlockSpec((_B, O), lambda i: (i, 0)),
        out_shape=jax.ShapeDtypeStruct((N, O), jnp.float32),
    )(opeps, z, q0, q1, ba, Wb, bb, Wout, bout)


def kernel(x, edge_index, eps0, W0a, b0a, W0b, b0b, g0, be0,
           eps1, W1a, b1a, W1b, b1b, g1, be1, Wout, bout):
    N = x.shape[0]
    src = edge_index[0]
    dst = edge_index[1]
    zrow = jnp.zeros((N // _NS, W0a.shape[1]), jnp.float32)

    bn = 1.0 / jnp.sqrt(jnp.float32(1.0 + 1e-5))
    s0 = g0 * bn
    W0bf = W0b * s0[None, :]
    b0bf = (b0b * s0 + be0)[None, :]
    s1 = g1 * bn
    W1bf = W1b * s1[None, :]
    b1bf = (b1b * s1 + be1)[None, :]

    ope0 = jnp.reshape(1.0 + eps0, (1,))
    ope1 = jnp.reshape(1.0 + eps1, (1,))

    y0 = _mm_tc(x, W0a)
    parts0 = _seg_sum_sc(y0, src, dst, zrow)
    p0a = parts0[:N]
    p0b = parts0[N:]
    z = _mid_tc(ope0, y0, p0a, p0b, b0a[None, :], W0bf, b0bf, W1a)

    parts1 = _seg_sum_sc(z, src, dst, zrow)
    q0 = parts1[:N]
    q1 = parts1[N:]
    return _head_tc(ope1, z, q0, q1, b1a[None, :], W1bf, b1bf,
                    Wout, bout[None, :])


# speedup vs baseline: 6.8604x; 6.8604x over previous
import functools

import jax
import jax.numpy as jnp
from jax import lax
from jax.experimental import pallas as pl
from jax.experimental.pallas import tpu as pltpu
from jax.experimental.pallas import tpu_sc as plsc

_NC = 2
_NS = 16
_C = 128
_B = 1000


def _seg_sum_sc(y, src, dst, zrow):
    N, F = y.shape
    E = src.shape[0]
    num_chunks = E // _C
    npad = -(-N // (8 * _NS)) * (8 * _NS)
    rpt = npad // _NS
    cpt = -(-num_chunks // (_NC * _NS))
    mesh = plsc.VectorSubcoreMesh(core_axis_name="c", subcore_axis_name="s")

    @functools.partial(
        pl.kernel,
        out_type=jax.ShapeDtypeStruct((_NC * npad, F), jnp.float32),
        mesh=mesh,
        scratch_types=[
            pltpu.VMEM((_C,), jnp.int32),
            pltpu.VMEM((_C,), jnp.int32),
            pltpu.VMEM((_C, F), jnp.float32),
            pltpu.VMEM_SHARED((npad, F), jnp.float32),
            pltpu.SemaphoreType.DMA,
        ],
        compiler_params=pltpu.CompilerParams(use_tc_tiling_on_sc=False),
    )
    def agg(y_hbm, src_hbm, dst_hbm, z_hbm, out_hbm, src_v, dst_v, rows_v,
            acc_sh, sem):
        cid = lax.axis_index("c")
        sid = lax.axis_index("s")
        wid = sid * _NC + cid
        row0 = sid * rpt
        pltpu.sync_copy(z_hbm, acc_sh.at[pl.ds(row0, rpt)])
        plsc.subcore_barrier()
        start_chunk = wid * cpt

        def body(j, carry):
            chunk = start_chunk + j

            @pl.when(chunk < num_chunks)
            def _():
                base = chunk * _C
                pltpu.sync_copy(src_hbm.at[pl.ds(base, _C)], src_v)
                pltpu.sync_copy(dst_hbm.at[pl.ds(base, _C)], dst_v)
                pltpu.async_copy(y_hbm.at[src_v], rows_v, sem).wait()
                pltpu.sync_copy(rows_v, acc_sh.at[dst_v], add=True)

            return carry

        lax.fori_loop(0, cpt, body, 0)
        plsc.subcore_barrier()
        pltpu.sync_copy(acc_sh.at[pl.ds(row0, rpt)],
                        out_hbm.at[pl.ds(cid * npad + row0, rpt)])

    return agg(y, src, dst, zrow), npad


def _mm_tc(x, W):
    N, D = x.shape
    H = W.shape[1]

    def body(x_ref, w_ref, o_ref):
        o_ref[...] = jnp.dot(x_ref[...], w_ref[...],
                             preferred_element_type=jnp.float32)

    return pl.pallas_call(
        body,
        grid=(N // _B,),
        in_specs=[
            pl.BlockSpec((_B, D), lambda i: (i, 0)),
            pl.BlockSpec((D, H), lambda i: (0, 0)),
        ],
        out_specs=pl.BlockSpec((_B, H), lambda i: (i, 0)),
        out_shape=jax.ShapeDtypeStruct((N, H), jnp.float32),
    )(x, W)


def _mid_tc(opeps, y, p0, p1, ba, Wb, bb, Wnext):
    N, H = y.shape
    H2 = Wnext.shape[1]

    def body(e_ref, y_ref, p0_ref, p1_ref, ba_ref, wb_ref, bb_ref, wn_ref,
             o_ref):
        e = e_ref[0]
        t = jnp.maximum(e * y_ref[...] + p0_ref[...] + p1_ref[...]
                        + ba_ref[...], 0.0)
        h = jnp.dot(t, wb_ref[...], preferred_element_type=jnp.float32)
        h = jnp.maximum(h + bb_ref[...], 0.0)
        o_ref[...] = jnp.dot(h, wn_ref[...],
                             preferred_element_type=jnp.float32)

    return pl.pallas_call(
        body,
        grid=(N // _B,),
        in_specs=[
            pl.BlockSpec(memory_space=pltpu.SMEM),
            pl.BlockSpec((_B, H), lambda i: (i, 0)),
            pl.BlockSpec((_B, H), lambda i: (i, 0)),
            pl.BlockSpec((_B, H), lambda i: (i, 0)),
            pl.BlockSpec((1, H), lambda i: (0, 0)),
            pl.BlockSpec((H, H), lambda i: (0, 0)),
            pl.BlockSpec((1, H), lambda i: (0, 0)),
            pl.BlockSpec((H, H2), lambda i: (0, 0)),
        ],
        out_specs=pl.BlockSpec((_B, H2), lambda i: (i, 0)),
        out_shape=jax.ShapeDtypeStruct((N, H2), jnp.float32),
    )(opeps, y, p0, p1, ba, Wb, bb, Wnext)


def _head_tc(opeps, z, q0, q1, ba, Wb, bb, Wout, bout):
    N, H = z.shape
    O = Wout.shape[1]

    def body(e_ref, z_ref, q0_ref, q1_ref, ba_ref, wb_ref, bb_ref, wo_ref,
             bo_ref, o_ref):
        e = e_ref[0]
        t = jnp.maximum(e * z_ref[...] + q0_ref[...] + q1_ref[...]
                        + ba_ref[...], 0.0)
        h = jnp.dot(t, wb_ref[...], preferred_element_type=jnp.float32)
        h = jnp.maximum(h + bb_ref[...], 0.0)
        logits = jnp.dot(h, wo_ref[...],
                         preferred_element_type=jnp.float32) + bo_ref[...]
        m = jnp.max(logits, axis=-1, keepdims=True)
        s = logits - m
        lse = jnp.log(jnp.sum(jnp.exp(s), axis=-1, keepdims=True))
        o_ref[...] = s - lse

    return pl.pallas_call(
        body,
        grid=(N // _B,),
        in_specs=[
            pl.BlockSpec(memory_space=pltpu.SMEM),
            pl.BlockSpec((_B, H), lambda i: (i, 0)),
            pl.BlockSpec((_B, H), lambda i: (i, 0)),
            pl.BlockSpec((_B, H), lambda i: (i, 0)),
            pl.BlockSpec((1, H), lambda i: (0, 0)),
            pl.BlockSpec((H, H), lambda i: (0, 0)),
            pl.BlockSpec((1, H), lambda i: (0, 0)),
            pl.BlockSpec((H, O), lambda i: (0, 0)),
            pl.BlockSpec((1, O), lambda i: (0, 0)),
        ],
        out_specs=pl.BlockSpec((_B, O), lambda i: (i, 0)),
        out_shape=jax.ShapeDtypeStruct((N, O), jnp.float32),
    )(opeps, z, q0, q1, ba, Wb, bb, Wout, bout)


def kernel(x, edge_index, eps0, W0a, b0a, W0b, b0b, g0, be0,
           eps1, W1a, b1a, W1b, b1b, g1, be1, Wout, bout):
    N = x.shape[0]
    src = edge_index[0]
    dst = edge_index[1]
    npad = -(-N // (8 * _NS)) * (8 * _NS)
    zrow = jnp.zeros((npad // _NS, W0a.shape[1]), jnp.float32)

    bn = 1.0 / jnp.sqrt(jnp.float32(1.0 + 1e-5))
    s0 = g0 * bn
    W0bf = W0b * s0[None, :]
    b0bf = (b0b * s0 + be0)[None, :]
    s1 = g1 * bn
    W1bf = W1b * s1[None, :]
    b1bf = (b1b * s1 + be1)[None, :]

    ope0 = jnp.reshape(1.0 + eps0, (1,))
    ope1 = jnp.reshape(1.0 + eps1, (1,))

    y0 = _mm_tc(x, W0a)
    parts0, npad = _seg_sum_sc(y0, src, dst, zrow)
    p0a = parts0[:N]
    p0b = parts0[npad:npad + N]
    z = _mid_tc(ope0, y0, p0a, p0b, b0a[None, :], W0bf, b0bf, W1a)

    parts1, npad = _seg_sum_sc(z, src, dst, zrow)
    q0 = parts1[:N]
    q1 = parts1[npad:npad + N]
    return _head_tc(ope1, z, q0, q1, b1a[None, :], W1bf, b1bf,
                    Wout, bout[None, :])
